# baseline (speedup 1.0000x reference)
import jax
import jax.numpy as jnp
from jax.experimental import pallas as pl
from jax.experimental.pallas import tpu as pltpu

_C = 80
_R = 2048
_B = 256
_S_BLK = 16
_ROWS = _S_BLK * _C
_STEPS = _B // _S_BLK


def _ce_kernel(lab_ref, x_ref, w_ref, loss_ref, acc_ref, corr_ref, num_ref):
    i = pl.program_id(0)

    @pl.when(i == 0)
    def _init():
        loss_ref[0, 0] = 0.0
        corr_ref[0] = 0.0
        num_ref[0] = 0.0

    x = x_ref[...]
    w = w_ref[...]
    lt = jax.lax.dot_general(
        w, x, (((1,), (1,)), ((), ())),
        preferred_element_type=jnp.float32)

    m = jnp.max(lt, axis=0, keepdims=True)
    e = jnp.exp(lt - m)
    lse = jnp.log(jnp.sum(e, axis=0, keepdims=True)) + m

    lane = jax.lax.broadcasted_iota(jnp.int32, (1, _ROWS), 1)
    cvec = lane % _C
    seg = lane // _C
    krows = jax.lax.broadcasted_iota(jnp.int32, (_C, _ROWS), 0)
    diag = jnp.sum(jnp.where(krows == cvec, lt, 0.0), axis=0, keepdims=True)
    ce = lse - diag
    idx = jnp.min(jnp.where(lt == m, krows, _C), axis=0, keepdims=True)

    maskf = (lab_ref[...] > 0).astype(jnp.float32)
    inv = jnp.zeros((1, _ROWS), jnp.float32)
    for s in range(_S_BLK):
        ns = jnp.sum(maskf[0, s * _C:(s + 1) * _C])
        inv = jnp.where(seg == s, 1.0 / (jnp.maximum(ns, 1.0) * _B), inv)

    loss_ref[0, 0] += jnp.sum(ce * maskf * inv)
    corr_ref[0] += jnp.sum(jnp.where(idx == cvec, maskf, 0.0))
    num_ref[0] += jnp.sum(maskf)

    @pl.when(i == _STEPS - 1)
    def _fin():
        acc_ref[0, 0] = corr_ref[0] / num_ref[0]


def _run(x, label, W):
    x2 = x.reshape(_B * _C, _R)
    labf = label.reshape(1, _B * _C)
    loss, acc = pl.pallas_call(
        _ce_kernel,
        grid=(_STEPS,),
        in_specs=[
            pl.BlockSpec((1, _ROWS), lambda i: (0, i)),
            pl.BlockSpec((_ROWS, _R), lambda i: (i, 0)),
            pl.BlockSpec((_C, _R), lambda i: (0, 0)),
        ],
        out_specs=[
            pl.BlockSpec(memory_space=pltpu.SMEM),
            pl.BlockSpec(memory_space=pltpu.SMEM),
        ],
        out_shape=[
            jax.ShapeDtypeStruct((1, 1), jnp.float32),
            jax.ShapeDtypeStruct((1, 1), jnp.float32),
        ],
        scratch_shapes=[
            pltpu.SMEM((1,), jnp.float32),
            pltpu.SMEM((1,), jnp.float32),
        ],
        compiler_params=pltpu.CompilerParams(
            dimension_semantics=("arbitrary",)),
    )(labf, x2, W)
    return loss.reshape(()), acc.reshape(())


def kernel(x, label, W):
    return _run(x, label, W)
